# baseline (speedup 1.0000x reference)
import jax
import jax.numpy as jnp
from jax.experimental import pallas as pl

NUM_CATEGORIES = 1000
ROW_BLOCK = 2048


def _onehot_body(inp_ref, out_ref):
    inp = inp_ref[...]
    iota = jax.lax.broadcasted_iota(jnp.int32, (ROW_BLOCK, NUM_CATEGORIES), 1)
    out_ref[...] = (iota == inp[:, None]).astype(jnp.float32)


def kernel(inputs):
    batch, nfeat = inputs.shape
    n = batch * nfeat
    flat = inputs.astype(jnp.int32).reshape(n)
    out2d = pl.pallas_call(
        _onehot_body,
        grid=(n // ROW_BLOCK,),
        in_specs=[pl.BlockSpec((ROW_BLOCK,), lambda i: (i,))],
        out_specs=pl.BlockSpec((ROW_BLOCK, NUM_CATEGORIES), lambda i: (i, 0)),
        out_shape=jax.ShapeDtypeStruct((n, NUM_CATEGORIES), jnp.float32),
    )(flat)
    return out2d.reshape(batch, nfeat, NUM_CATEGORIES)
